# baseline (speedup 1.0000x reference)
import functools

import jax
import jax.numpy as jnp
from jax import lax
from jax.experimental import pallas as pl
from jax.experimental.pallas import tpu as pltpu
from jax.experimental.pallas import tpu_sc as plsc

_N = 10000
_NPAD = 10240
_ROWW = 144
_C = 128
_NTILES = 32
_E = 330000
_TCHUNK = 82
_EPAD = _NTILES * _C * _TCHUNK
_R = 512
_SUB = 16
_ZR = 64


def _sc_agg(heads, g, ed, srcp, dstp):
    rows_per_tile = _NPAD // _SUB
    vregs_per_head = 8 // heads

    mesh = plsc.VectorSubcoreMesh(core_axis_name="c", subcore_axis_name="s")

    @functools.partial(
        pl.kernel,
        out_type=jax.ShapeDtypeStruct((2, _NPAD, _ROWW), jnp.float32),
        mesh=mesh,
        compiler_params=pltpu.CompilerParams(use_tc_tiling_on_sc=False),
        scratch_types=[
            pltpu.VMEM((_C,), jnp.int32),
            pltpu.VMEM((_C,), jnp.int32),
            pltpu.VMEM((_C, _ROWW), jnp.float32),
            pltpu.VMEM((_C, 16), jnp.float32),
            pltpu.VMEM((_ZR, _ROWW), jnp.float32),
            pltpu.VMEM_SHARED((_NPAD, _ROWW), jnp.float32),
            pltpu.SemaphoreType.DMA,
            pltpu.SemaphoreType.DMA,
        ],
    )
    def k(g_hbm, ed_hbm, src_hbm, dst_hbm, out_hbm,
          src_v, dst_v, rows_v, edr_v, zb_v, acc_sh, sem1, sem2):
        cid = lax.axis_index("c")
        sid = lax.axis_index("s")
        wid = sid * 2 + cid
        lane = lax.iota(jnp.int32, 16)

        def zrow(r, c):
            for j in range(_ROWW // 16):
                zb_v[r, pl.ds(16 * j, 16)] = jnp.zeros((16,), jnp.float32)
            return c
        lax.fori_loop(0, _ZR, zrow, 0)

        def zcp(t, c):
            pltpu.sync_copy(
                zb_v, acc_sh.at[pl.ds(sid * rows_per_tile + t * _ZR, _ZR)])
            return c
        lax.fori_loop(0, rows_per_tile // _ZR, zcp, 0)

        plsc.subcore_barrier()

        def compute():
            @plsc.parallel_loop(0, _C, unroll=4)
            def edge(i):
                ves = rows_v[i, pl.ds(128, 16)]
                ved = edr_v[i, pl.ds(0, 16)]
                w = ves + ved
                w = jnp.where(w > 0, w, 0.2 * w)
                p = jnp.exp(w)
                p = jnp.where(lane < heads, p, 0.0)
                rows_v[i, pl.ds(128, 16)] = p
                for h in range(heads):
                    scale = lax.gather(
                        p, jnp.full((16, 1), h, jnp.int32),
                        lax.GatherDimensionNumbers(
                            offset_dims=(), collapsed_slice_dims=(0,),
                            start_index_map=(0,)),
                        (1,),
                        mode=lax.GatherScatterMode.PROMISE_IN_BOUNDS)
                    for jj in range(vregs_per_head):
                        j = h * vregs_per_head + jj
                        v = rows_v[i, pl.ds(16 * j, 16)]
                        rows_v[i, pl.ds(16 * j, 16)] = v * scale

        def chunk(t, c):
            base = (wid * _TCHUNK + t) * _C
            pltpu.sync_copy(src_hbm.at[pl.ds(base, _C)], src_v)
            pltpu.sync_copy(dst_hbm.at[pl.ds(base, _C)], dst_v)
            pltpu.async_copy(g_hbm.at[src_v], rows_v, sem1).wait()
            pltpu.async_copy(ed_hbm.at[dst_v], edr_v, sem2).wait()
            compute()
            pltpu.sync_copy(rows_v, acc_sh.at[dst_v], add=True)
            return c
        lax.fori_loop(0, _TCHUNK, chunk, 0)
        plsc.subcore_barrier()

        def ocp(t, c):
            off = sid * rows_per_tile + t * _ZR
            pltpu.sync_copy(acc_sh.at[pl.ds(off, _ZR)],
                            out_hbm.at[cid, pl.ds(off, _ZR)])
            return c
        lax.fori_loop(0, rows_per_tile // _ZR, ocp, 0)

    return k(g, ed, srcp, dstp)


def _pre_body(x_ref, wcat_ref, a_ref, b_ref, ws_ref, bs_ref,
              g_ref, ed_ref, skip_ref):
    xb = x_ref[...]
    h = jnp.dot(xb, wcat_ref[...], preferred_element_type=jnp.float32)
    es = jnp.dot(h, a_ref[...], preferred_element_type=jnp.float32)
    ed = jnp.dot(h, b_ref[...], preferred_element_type=jnp.float32)
    z12 = jnp.zeros((_R, 12), jnp.float32)
    g_ref[...] = jnp.concatenate([h, es, z12], axis=1)
    ed_ref[...] = jnp.concatenate([ed, z12], axis=1)
    skip_ref[...] = (
        jnp.dot(xb, ws_ref[...], preferred_element_type=jnp.float32)
        + bs_ref[...])


def _mid_body(p_ref, bc_ref, wl_ref, asl_ref, adl_ref, g2_ref, ed2_ref):
    num = p_ref[0, :, :128] + p_ref[1, :, :128]
    den = p_ref[0, :, 128:132] + p_ref[1, :, 128:132]
    denb = jnp.reshape(
        jnp.broadcast_to(den[:, :, None], (_R, 4, 32)), (_R, 128))
    h2 = num / (denb + 1e-16) + bc_ref[...]
    hh = jnp.dot(h2, wl_ref[...], preferred_element_type=jnp.float32)
    es2 = jnp.dot(hh, asl_ref[...], preferred_element_type=jnp.float32)
    ed2 = jnp.dot(hh, adl_ref[...], preferred_element_type=jnp.float32)
    z15 = jnp.zeros((_R, 15), jnp.float32)
    g2_ref[...] = jnp.concatenate([hh, es2, z15], axis=1)
    ed2_ref[...] = jnp.concatenate([ed2, z15], axis=1)


def _fin_body(p_ref, skip_ref, bl_ref, o_ref):
    num = p_ref[0, :, :128] + p_ref[1, :, :128]
    den = p_ref[0, :, 128:129] + p_ref[1, :, 128:129]
    y = num / (den + 1e-16) + bl_ref[...] + skip_ref[...]
    m = jnp.max(y, axis=1, keepdims=True)
    z = y - m
    o_ref[...] = z - jnp.log(jnp.sum(jnp.exp(z), axis=1, keepdims=True))


def _full(shape):
    return pl.BlockSpec(shape, lambda i: tuple(0 for _ in shape))


def kernel(x, edge_index, Wc, asrc, adst, bc, Wl, asl, adl, bl, Ws, bs):
    f32 = jnp.float32
    grid = _NPAD // _R

    x_pad = jnp.zeros((_NPAD, 128), f32).at[:_N].set(x)
    Wcat = Wc.transpose(1, 0, 2).reshape(128, 128)
    eye = jnp.eye(4, dtype=f32)
    A = (asrc[:, :, None] * eye[:, None, :]).reshape(128, 4)
    B = (adst[:, :, None] * eye[:, None, :]).reshape(128, 4)

    loop = jnp.arange(_N, dtype=jnp.int32)
    pad = jnp.full((_EPAD - _E,), _N, jnp.int32)
    srcp = jnp.concatenate([edge_index[0], loop, pad])
    dstp = jnp.concatenate([edge_index[1], loop, pad])

    g1, ed1, xskip = pl.pallas_call(
        _pre_body,
        grid=(grid,),
        in_specs=[
            pl.BlockSpec((_R, 128), lambda i: (i, 0)),
            _full((128, 128)), _full((128, 4)), _full((128, 4)),
            _full((128, 128)), _full((1, 128)),
        ],
        out_specs=[
            pl.BlockSpec((_R, _ROWW), lambda i: (i, 0)),
            pl.BlockSpec((_R, 16), lambda i: (i, 0)),
            pl.BlockSpec((_R, 128), lambda i: (i, 0)),
        ],
        out_shape=[
            jax.ShapeDtypeStruct((_NPAD, _ROWW), f32),
            jax.ShapeDtypeStruct((_NPAD, 16), f32),
            jax.ShapeDtypeStruct((_NPAD, 128), f32),
        ],
    )(x_pad, Wcat, A, B, Ws, bs[None, :])

    parts1 = _sc_agg(4, g1, ed1, srcp, dstp)

    g2, ed2 = pl.pallas_call(
        _mid_body,
        grid=(grid,),
        in_specs=[
            pl.BlockSpec((2, _R, _ROWW), lambda i: (0, i, 0)),
            _full((1, 128)), _full((128, 128)),
            _full((128, 1)), _full((128, 1)),
        ],
        out_specs=[
            pl.BlockSpec((_R, _ROWW), lambda i: (i, 0)),
            pl.BlockSpec((_R, 16), lambda i: (i, 0)),
        ],
        out_shape=[
            jax.ShapeDtypeStruct((_NPAD, _ROWW), f32),
            jax.ShapeDtypeStruct((_NPAD, 16), f32),
        ],
    )(parts1, bc.reshape(1, 128), Wl, asl[:, None], adl[:, None])

    parts2 = _sc_agg(1, g2, ed2, srcp, dstp)

    out = pl.pallas_call(
        _fin_body,
        grid=(grid,),
        in_specs=[
            pl.BlockSpec((2, _R, _ROWW), lambda i: (0, i, 0)),
            pl.BlockSpec((_R, 128), lambda i: (i, 0)),
            _full((1, 128)),
        ],
        out_specs=pl.BlockSpec((_R, 128), lambda i: (i, 0)),
        out_shape=jax.ShapeDtypeStruct((_NPAD, 128), f32),
    )(parts2, xskip, bl[None, :])

    return out[:_N]


# speedup vs baseline: 1.6491x; 1.6491x over previous
import functools

import jax
import jax.numpy as jnp
from jax import lax
from jax.experimental import pallas as pl
from jax.experimental.pallas import tpu as pltpu
from jax.experimental.pallas import tpu_sc as plsc

_N = 10000
_NPAD = 10240
_ROWW = 144
_C = 128
_NTILES = 32
_E = 330000
_TCHUNK = 82
_EPAD = _NTILES * _C * _TCHUNK
_R = 512
_SUB = 16
_ZR = 64


def _sc_agg(heads, g, ed, srcp, dstp):
    rows_per_tile = _NPAD // _SUB
    vregs_per_head = 8 // heads

    mesh = plsc.VectorSubcoreMesh(core_axis_name="c", subcore_axis_name="s")

    @functools.partial(
        pl.kernel,
        out_type=jax.ShapeDtypeStruct((2, _NPAD, _ROWW), jnp.float32),
        mesh=mesh,
        compiler_params=pltpu.CompilerParams(use_tc_tiling_on_sc=False),
        scratch_types=[
            pltpu.VMEM((_C,), jnp.int32),
            pltpu.VMEM((_C,), jnp.int32),
            pltpu.VMEM((_C, _ROWW), jnp.float32),
            pltpu.VMEM((_C, 16), jnp.float32),
            pltpu.VMEM((_ZR, _ROWW), jnp.float32),
            pltpu.VMEM_SHARED((_NPAD, _ROWW), jnp.float32),
            pltpu.SemaphoreType.DMA,
            pltpu.SemaphoreType.DMA,
        ],
    )
    def k(g_hbm, ed_hbm, src_hbm, dst_hbm, out_hbm,
          src_v, dst_v, rows_v, edr_v, zb_v, acc_sh, sem1, sem2):
        cid = lax.axis_index("c")
        sid = lax.axis_index("s")
        wid = sid * 2 + cid
        lane = lax.iota(jnp.int32, 16)

        def zrow(r, c):
            for j in range(_ROWW // 16):
                zb_v[r, pl.ds(16 * j, 16)] = jnp.zeros((16,), jnp.float32)
            return c
        lax.fori_loop(0, _ZR, zrow, 0)

        def zcp(t, c):
            pltpu.sync_copy(
                zb_v, acc_sh.at[pl.ds(sid * rows_per_tile + t * _ZR, _ZR)])
            return c
        lax.fori_loop(0, rows_per_tile // _ZR, zcp, 0)

        plsc.subcore_barrier()

        def compute():
            @plsc.parallel_loop(0, _C, unroll=4)
            def edge(i):
                ves = rows_v[i, pl.ds(128, 16)]
                ved = edr_v[i, pl.ds(0, 16)]
                w = ves + ved
                w = jnp.where(w > 0, w, 0.2 * w)
                p = jnp.exp(w)
                p = jnp.where(lane < heads, p, 0.0)
                rows_v[i, pl.ds(128, 16)] = p
                for h in range(heads):
                    scale = lax.gather(
                        p, jnp.full((16, 1), h, jnp.int32),
                        lax.GatherDimensionNumbers(
                            offset_dims=(), collapsed_slice_dims=(0,),
                            start_index_map=(0,)),
                        (1,),
                        mode=lax.GatherScatterMode.PROMISE_IN_BOUNDS)
                    for jj in range(vregs_per_head):
                        j = h * vregs_per_head + jj
                        v = rows_v[i, pl.ds(16 * j, 16)]
                        rows_v[i, pl.ds(16 * j, 16)] = v * scale

        def chunk(t, c):
            base = (wid * _TCHUNK + t) * _C
            pltpu.sync_copy(src_hbm.at[pl.ds(base, _C)], src_v)
            pltpu.sync_copy(dst_hbm.at[pl.ds(base, _C)], dst_v)
            pltpu.async_copy(g_hbm.at[src_v], rows_v, sem1).wait()
            pltpu.async_copy(ed_hbm.at[dst_v], edr_v, sem2).wait()
            compute()
            pltpu.sync_copy(rows_v, acc_sh.at[dst_v], add=True)
            return c
        lax.fori_loop(0, _TCHUNK, chunk, 0)
        plsc.subcore_barrier()

        def ocp(t, c):
            off = sid * rows_per_tile + t * _ZR
            pltpu.sync_copy(acc_sh.at[pl.ds(off, _ZR)],
                            out_hbm.at[cid, pl.ds(off, _ZR)])
            return c
        lax.fori_loop(0, rows_per_tile // _ZR, ocp, 0)

    return k(g, ed, srcp, dstp)


def _pre_body(x_ref, wcat_ref, a_ref, b_ref, ws_ref, bs_ref,
              g_ref, ed_ref, skip_ref):
    xb = x_ref[...]
    h = jnp.dot(xb, wcat_ref[...], preferred_element_type=jnp.float32)
    es = jnp.dot(h, a_ref[...], preferred_element_type=jnp.float32)
    ed = jnp.dot(h, b_ref[...], preferred_element_type=jnp.float32)
    z12 = jnp.zeros((_R, 12), jnp.float32)
    g_ref[...] = jnp.concatenate([h, es, z12], axis=1)
    ed_ref[...] = jnp.concatenate([ed, z12], axis=1)
    skip_ref[...] = (
        jnp.dot(xb, ws_ref[...], preferred_element_type=jnp.float32)
        + bs_ref[...])


def _mid_body(p_ref, bc_ref, wl_ref, asl_ref, adl_ref, g2_ref, ed2_ref):
    num = p_ref[0, :, :128] + p_ref[1, :, :128]
    den = p_ref[0, :, 128:132] + p_ref[1, :, 128:132]
    denb = jnp.reshape(
        jnp.broadcast_to(den[:, :, None], (_R, 4, 32)), (_R, 128))
    h2 = num / (denb + 1e-16) + bc_ref[...]
    hh = jnp.dot(h2, wl_ref[...], preferred_element_type=jnp.float32)
    es2 = jnp.dot(hh, asl_ref[...], preferred_element_type=jnp.float32)
    ed2 = jnp.dot(hh, adl_ref[...], preferred_element_type=jnp.float32)
    z15 = jnp.zeros((_R, 15), jnp.float32)
    g2_ref[...] = jnp.concatenate([hh, es2, z15], axis=1)
    ed2_ref[...] = jnp.concatenate([ed2, z15], axis=1)


def _fin_body(p_ref, skip_ref, bl_ref, o_ref):
    num = p_ref[0, :, :128] + p_ref[1, :, :128]
    den = p_ref[0, :, 128:129] + p_ref[1, :, 128:129]
    y = num / (den + 1e-16) + bl_ref[...] + skip_ref[...]
    m = jnp.max(y, axis=1, keepdims=True)
    z = y - m
    o_ref[...] = z - jnp.log(jnp.sum(jnp.exp(z), axis=1, keepdims=True))


def _full(shape):
    return pl.BlockSpec(shape, lambda i: tuple(0 for _ in shape))


def kernel(x, edge_index, Wc, asrc, adst, bc, Wl, asl, adl, bl, Ws, bs):
    f32 = jnp.float32
    grid = _NPAD // _R

    x_pad = jnp.zeros((_NPAD, 128), f32).at[:_N].set(x)
    Wcat = Wc.transpose(1, 0, 2).reshape(128, 128)
    eye = jnp.eye(4, dtype=f32)
    A = (asrc[:, :, None] * eye[:, None, :]).reshape(128, 4)
    B = (adst[:, :, None] * eye[:, None, :]).reshape(128, 4)

    loop = jnp.arange(_N, dtype=jnp.int32)
    pad = _N + jnp.arange(_EPAD - _E, dtype=jnp.int32) % (_NPAD - _N)
    srcp = jnp.concatenate([edge_index[0], loop, pad])
    dstp = jnp.concatenate([edge_index[1], loop, pad])

    g1, ed1, xskip = pl.pallas_call(
        _pre_body,
        grid=(grid,),
        in_specs=[
            pl.BlockSpec((_R, 128), lambda i: (i, 0)),
            _full((128, 128)), _full((128, 4)), _full((128, 4)),
            _full((128, 128)), _full((1, 128)),
        ],
        out_specs=[
            pl.BlockSpec((_R, _ROWW), lambda i: (i, 0)),
            pl.BlockSpec((_R, 16), lambda i: (i, 0)),
            pl.BlockSpec((_R, 128), lambda i: (i, 0)),
        ],
        out_shape=[
            jax.ShapeDtypeStruct((_NPAD, _ROWW), f32),
            jax.ShapeDtypeStruct((_NPAD, 16), f32),
            jax.ShapeDtypeStruct((_NPAD, 128), f32),
        ],
    )(x_pad, Wcat, A, B, Ws, bs[None, :])

    parts1 = _sc_agg(4, g1, ed1, srcp, dstp)

    g2, ed2 = pl.pallas_call(
        _mid_body,
        grid=(grid,),
        in_specs=[
            pl.BlockSpec((2, _R, _ROWW), lambda i: (0, i, 0)),
            _full((1, 128)), _full((128, 128)),
            _full((128, 1)), _full((128, 1)),
        ],
        out_specs=[
            pl.BlockSpec((_R, _ROWW), lambda i: (i, 0)),
            pl.BlockSpec((_R, 16), lambda i: (i, 0)),
        ],
        out_shape=[
            jax.ShapeDtypeStruct((_NPAD, _ROWW), f32),
            jax.ShapeDtypeStruct((_NPAD, 16), f32),
        ],
    )(parts1, bc.reshape(1, 128), Wl, asl[:, None], adl[:, None])

    parts2 = _sc_agg(1, g2, ed2, srcp, dstp)

    out = pl.pallas_call(
        _fin_body,
        grid=(grid,),
        in_specs=[
            pl.BlockSpec((2, _R, _ROWW), lambda i: (0, i, 0)),
            pl.BlockSpec((_R, 128), lambda i: (i, 0)),
            _full((1, 128)),
        ],
        out_specs=pl.BlockSpec((_R, 128), lambda i: (i, 0)),
        out_shape=jax.ShapeDtypeStruct((_NPAD, 128), f32),
    )(parts2, xskip, bl[None, :])

    return out[:_N]


# speedup vs baseline: 1.8754x; 1.1372x over previous
import functools

import jax
import jax.numpy as jnp
from jax import lax
from jax.experimental import pallas as pl
from jax.experimental.pallas import tpu as pltpu
from jax.experimental.pallas import tpu_sc as plsc

_N = 10000
_NPAD = 10240
_ROWW = 144
_C = 128
_NTILES = 32
_E = 330000
_TCHUNK = 82
_EPAD = _NTILES * _C * _TCHUNK
_R = 512
_SUB = 16
_ZR = 64


def _sc_agg(heads, g, ed, srcp, dstp):
    rows_per_tile = _NPAD // _SUB
    vregs_per_head = 8 // heads

    mesh = plsc.VectorSubcoreMesh(core_axis_name="c", subcore_axis_name="s")

    @functools.partial(
        pl.kernel,
        out_type=jax.ShapeDtypeStruct((2, _NPAD, _ROWW), jnp.float32),
        mesh=mesh,
        compiler_params=pltpu.CompilerParams(use_tc_tiling_on_sc=False),
        scratch_types=[
            pltpu.VMEM((_C,), jnp.int32),
            pltpu.VMEM((_C,), jnp.int32),
            pltpu.VMEM((_C, _ROWW), jnp.float32),
            pltpu.VMEM((_C, 16), jnp.float32),
            pltpu.VMEM((_ZR, _ROWW), jnp.float32),
            pltpu.VMEM_SHARED((_NPAD, _ROWW), jnp.float32),
            pltpu.SemaphoreType.DMA,
            pltpu.SemaphoreType.DMA,
        ],
    )
    def k(g_hbm, ed_hbm, src_hbm, dst_hbm, out_hbm,
          src_v, dst_v, rows_v, edr_v, zb_v, acc_sh, sem1, sem2):
        cid = lax.axis_index("c")
        sid = lax.axis_index("s")
        wid = sid * 2 + cid
        lane = lax.iota(jnp.int32, 16)

        def zrow(r, c):
            for j in range(_ROWW // 16):
                zb_v[r, pl.ds(16 * j, 16)] = jnp.zeros((16,), jnp.float32)
            return c
        lax.fori_loop(0, _ZR, zrow, 0)

        def zcp(t, c):
            pltpu.sync_copy(
                zb_v, acc_sh.at[pl.ds(sid * rows_per_tile + t * _ZR, _ZR)])
            return c
        lax.fori_loop(0, rows_per_tile // _ZR, zcp, 0)

        plsc.subcore_barrier()

        def compute():
            @plsc.parallel_loop(0, _C, unroll=4)
            def edge(i):
                ves = rows_v[i, pl.ds(128, 16)]
                ved = edr_v[i, pl.ds(0, 16)]
                w = ves + ved
                w = jnp.where(w > 0, w, 0.2 * w)
                p = jnp.exp(w)
                p = jnp.where(lane < heads, p, 0.0)
                rows_v[i, pl.ds(128, 16)] = p
                for h in range(heads):
                    scale = lax.gather(
                        p, jnp.full((16, 1), h, jnp.int32),
                        lax.GatherDimensionNumbers(
                            offset_dims=(), collapsed_slice_dims=(0,),
                            start_index_map=(0,)),
                        (1,),
                        mode=lax.GatherScatterMode.PROMISE_IN_BOUNDS)
                    for jj in range(vregs_per_head):
                        j = h * vregs_per_head + jj
                        v = rows_v[i, pl.ds(16 * j, 16)]
                        rows_v[i, pl.ds(16 * j, 16)] = v * scale

        def chunk(t, c):
            base = (wid * _TCHUNK + t) * _C
            pltpu.sync_copy(src_hbm.at[pl.ds(base, _C)], src_v)
            pltpu.sync_copy(dst_hbm.at[pl.ds(base, _C)], dst_v)
            cg = pltpu.async_copy(g_hbm.at[src_v], rows_v, sem1)
            ce = pltpu.async_copy(ed_hbm.at[dst_v], edr_v, sem2)
            cg.wait()
            ce.wait()
            compute()
            pltpu.sync_copy(rows_v, acc_sh.at[dst_v], add=True)
            return c
        lax.fori_loop(0, _TCHUNK, chunk, 0)
        plsc.subcore_barrier()

        def ocp(t, c):
            off = sid * rows_per_tile + t * _ZR
            pltpu.sync_copy(acc_sh.at[pl.ds(off, _ZR)],
                            out_hbm.at[cid, pl.ds(off, _ZR)])
            return c
        lax.fori_loop(0, rows_per_tile // _ZR, ocp, 0)

    return k(g, ed, srcp, dstp)


def _pre_body(x_ref, wcat_ref, a_ref, b_ref, ws_ref, bs_ref,
              g_ref, ed_ref, skip_ref):
    xb = x_ref[...]
    h = jnp.dot(xb, wcat_ref[...], preferred_element_type=jnp.float32)
    es = jnp.dot(h, a_ref[...], preferred_element_type=jnp.float32)
    ed = jnp.dot(h, b_ref[...], preferred_element_type=jnp.float32)
    z12 = jnp.zeros((_R, 12), jnp.float32)
    g_ref[...] = jnp.concatenate([h, es, z12], axis=1)
    ed_ref[...] = jnp.concatenate([ed, z12], axis=1)
    skip_ref[...] = (
        jnp.dot(xb, ws_ref[...], preferred_element_type=jnp.float32)
        + bs_ref[...])


def _mid_body(p_ref, bc_ref, wl_ref, asl_ref, adl_ref, g2_ref, ed2_ref):
    num = p_ref[0, :, :128] + p_ref[1, :, :128]
    den = p_ref[0, :, 128:132] + p_ref[1, :, 128:132]
    denb = jnp.reshape(
        jnp.broadcast_to(den[:, :, None], (_R, 4, 32)), (_R, 128))
    h2 = num / (denb + 1e-16) + bc_ref[...]
    hh = jnp.dot(h2, wl_ref[...], preferred_element_type=jnp.float32)
    es2 = jnp.dot(hh, asl_ref[...], preferred_element_type=jnp.float32)
    ed2 = jnp.dot(hh, adl_ref[...], preferred_element_type=jnp.float32)
    z15 = jnp.zeros((_R, 15), jnp.float32)
    g2_ref[...] = jnp.concatenate([hh, es2, z15], axis=1)
    ed2_ref[...] = jnp.concatenate([ed2, z15], axis=1)


def _fin_body(p_ref, skip_ref, bl_ref, o_ref):
    num = p_ref[0, :, :128] + p_ref[1, :, :128]
    den = p_ref[0, :, 128:129] + p_ref[1, :, 128:129]
    y = num / (den + 1e-16) + bl_ref[...] + skip_ref[...]
    m = jnp.max(y, axis=1, keepdims=True)
    z = y - m
    o_ref[...] = z - jnp.log(jnp.sum(jnp.exp(z), axis=1, keepdims=True))


def _full(shape):
    return pl.BlockSpec(shape, lambda i: tuple(0 for _ in shape))


def kernel(x, edge_index, Wc, asrc, adst, bc, Wl, asl, adl, bl, Ws, bs):
    f32 = jnp.float32
    grid = _NPAD // _R

    x_pad = jnp.zeros((_NPAD, 128), f32).at[:_N].set(x)
    Wcat = Wc.transpose(1, 0, 2).reshape(128, 128)
    eye = jnp.eye(4, dtype=f32)
    A = (asrc[:, :, None] * eye[:, None, :]).reshape(128, 4)
    B = (adst[:, :, None] * eye[:, None, :]).reshape(128, 4)

    loop = jnp.arange(_N, dtype=jnp.int32)
    pad = _N + jnp.arange(_EPAD - _E, dtype=jnp.int32) % (_NPAD - _N)
    srcp = jnp.concatenate([edge_index[0], loop, pad])
    dstp = jnp.concatenate([edge_index[1], loop, pad])

    g1, ed1, xskip = pl.pallas_call(
        _pre_body,
        grid=(grid,),
        in_specs=[
            pl.BlockSpec((_R, 128), lambda i: (i, 0)),
            _full((128, 128)), _full((128, 4)), _full((128, 4)),
            _full((128, 128)), _full((1, 128)),
        ],
        out_specs=[
            pl.BlockSpec((_R, _ROWW), lambda i: (i, 0)),
            pl.BlockSpec((_R, 16), lambda i: (i, 0)),
            pl.BlockSpec((_R, 128), lambda i: (i, 0)),
        ],
        out_shape=[
            jax.ShapeDtypeStruct((_NPAD, _ROWW), f32),
            jax.ShapeDtypeStruct((_NPAD, 16), f32),
            jax.ShapeDtypeStruct((_NPAD, 128), f32),
        ],
    )(x_pad, Wcat, A, B, Ws, bs[None, :])

    parts1 = _sc_agg(4, g1, ed1, srcp, dstp)

    g2, ed2 = pl.pallas_call(
        _mid_body,
        grid=(grid,),
        in_specs=[
            pl.BlockSpec((2, _R, _ROWW), lambda i: (0, i, 0)),
            _full((1, 128)), _full((128, 128)),
            _full((128, 1)), _full((128, 1)),
        ],
        out_specs=[
            pl.BlockSpec((_R, _ROWW), lambda i: (i, 0)),
            pl.BlockSpec((_R, 16), lambda i: (i, 0)),
        ],
        out_shape=[
            jax.ShapeDtypeStruct((_NPAD, _ROWW), f32),
            jax.ShapeDtypeStruct((_NPAD, 16), f32),
        ],
    )(parts1, bc.reshape(1, 128), Wl, asl[:, None], adl[:, None])

    parts2 = _sc_agg(1, g2, ed2, srcp, dstp)

    out = pl.pallas_call(
        _fin_body,
        grid=(grid,),
        in_specs=[
            pl.BlockSpec((2, _R, _ROWW), lambda i: (0, i, 0)),
            pl.BlockSpec((_R, 128), lambda i: (i, 0)),
            _full((1, 128)),
        ],
        out_specs=pl.BlockSpec((_R, 128), lambda i: (i, 0)),
        out_shape=jax.ShapeDtypeStruct((_NPAD, 128), f32),
    )(parts2, xskip, bl[None, :])

    return out[:_N]
